# Initial kernel scaffold; baseline (speedup 1.0000x reference)
import functools

import jax
import jax.numpy as jnp
from jax import lax
from jax.experimental import pallas as pl
from jax.experimental.pallas import tpu as pltpu
from jax.experimental.pallas import tpu_sc as plsc

_NC = 2
_NS = 16
_NW = _NC * _NS

_CHUNK_ROWS = 64
_GW = 128


def _fm_body(idx_hbm, val_hbm, wlin_hbm, wemb_hbm, out_hbm,
             idx_v, val_v, emb_v, lin_v, out_v, sem,
             *, n_chunks, n_groups, fields, dim):
    wid = lax.axis_index("s") * _NC + lax.axis_index("c")

    def chunk_body(c, _):
        pltpu.sync_copy(idx_hbm.at[wid, c], idx_v)
        pltpu.sync_copy(val_hbm.at[wid, c], val_v)
        copies = []
        for j in range(n_groups):
            copies.append(pltpu.async_copy(
                wemb_hbm.at[idx_v.at[j]],
                emb_v.at[pl.ds(j * _GW, _GW)], sem))
            copies.append(pltpu.async_copy(
                wlin_hbm.at[idx_v.at[j]],
                lin_v.at[pl.ds(j * _GW, _GW)], sem))
        for cp in copies:
            cp.wait()

        def row_body(b, _):
            n0 = b * fields
            acc = jnp.zeros((dim,), jnp.float32)
            accq = jnp.zeros((dim,), jnp.float32)
            lin = jnp.float32(0.0)
            for f in range(fields):
                n = n0 + f
                e = emb_v[n, :]
                v = val_v[n]
                ev = e * v
                acc = acc + ev
                accq = accq + ev * ev
                lin = lin + lin_v[n] * v
            r = acc * acc - accq
            out_v[b] = 0.5 * jnp.sum(r) + lin
            return 0

        lax.fori_loop(0, _CHUNK_ROWS, row_body, 0, unroll=False)
        pltpu.sync_copy(out_v, out_hbm.at[wid, pl.ds(c * _CHUNK_ROWS,
                                                     _CHUNK_ROWS)])
        return 0

    lax.fori_loop(0, n_chunks, chunk_body, 0, unroll=False)


def kernel(feature_idx, feature_value, W_linear, bias, W_embed):
    batch, fields = feature_idx.shape
    dim = W_embed.shape[1]
    assert batch % (_NW * _CHUNK_ROWS) == 0
    rows_per_w = batch // _NW
    n_chunks = rows_per_w // _CHUNK_ROWS
    g = _CHUNK_ROWS * fields
    assert g % _GW == 0
    n_groups = g // _GW

    idx_r = feature_idx.reshape(_NW, n_chunks, n_groups, _GW)
    val_r = feature_value.reshape(_NW, n_chunks, g)
    wlin = W_linear.reshape(-1)

    mesh = plsc.VectorSubcoreMesh(core_axis_name="c", subcore_axis_name="s")
    body = functools.partial(_fm_body, n_chunks=n_chunks, n_groups=n_groups,
                             fields=fields, dim=dim)
    out = pl.kernel(
        body,
        out_type=jax.ShapeDtypeStruct((_NW, rows_per_w), jnp.float32),
        mesh=mesh,
        scratch_types=[
            pltpu.VMEM((n_groups, _GW), jnp.int32),
            pltpu.VMEM((g,), jnp.float32),
            pltpu.VMEM((g, dim), jnp.float32),
            pltpu.VMEM((g,), jnp.float32),
            pltpu.VMEM((_CHUNK_ROWS,), jnp.float32),
            pltpu.SemaphoreType.DMA,
        ],
    )(idx_r, val_r, wlin, W_embed)
    return out.reshape(batch, 1) + bias[None, :]


# speedup vs baseline: 1.2944x; 1.2944x over previous
import functools

import jax
import jax.numpy as jnp
from jax import lax
from jax.experimental import pallas as pl
from jax.experimental.pallas import tpu as pltpu
from jax.experimental.pallas import tpu_sc as plsc

_NC = 2
_NS = 16
_NW = _NC * _NS

_CHUNK_ROWS = 64
_GW = 128


def _fm_body(idx_hbm, val_hbm, wlin_hbm, wemb_hbm, out_hbm,
             idx_v, val_v, emb_v, lin_v, out_v, sem,
             *, n_chunks, n_groups, fields, dim):
    wid = lax.axis_index("s") * _NC + lax.axis_index("c")
    g = _CHUNK_ROWS * fields

    val_v[pl.ds(g, dim)] = jnp.zeros((dim,), jnp.float32)
    lin_v[pl.ds(g, dim)] = jnp.zeros((dim,), jnp.float32)

    def chunk_body(c, _):
        pltpu.sync_copy(idx_hbm.at[wid, c], idx_v)
        pltpu.sync_copy(val_hbm.at[wid, c], val_v.at[pl.ds(0, g)])
        copies = []
        for j in range(n_groups):
            copies.append(pltpu.async_copy(
                wemb_hbm.at[idx_v.at[j]],
                emb_v.at[pl.ds(j * _GW, _GW)], sem))
            copies.append(pltpu.async_copy(
                wlin_hbm.at[idx_v.at[j]],
                lin_v.at[pl.ds(j * _GW, _GW)], sem))
        for cp in copies:
            cp.wait()

        def row_body(b, _):
            lanes = lax.iota(jnp.int32, dim)
            last_lane = lanes == (dim - 1)
            lin_mask = lanes < (fields - 16)
            n0 = b * fields
            vv1 = val_v[pl.ds(n0, dim)]
            vv2 = val_v[pl.ds(n0 + dim, dim)]
            lv1 = lin_v[pl.ds(n0, dim)]
            lv2 = lin_v[pl.ds(n0 + dim, dim)]
            acc = jnp.zeros((dim,), jnp.float32)
            accq = jnp.zeros((dim,), jnp.float32)
            for f in range(fields):
                e = emb_v[n0 + f, :]
                v = vv1[f] if f < dim else vv2[f - dim]
                ev = e * v
                acc = acc + ev
                accq = accq + ev * ev
            r = (0.5 * (acc * acc - accq) + vv1 * lv1
                 + jnp.where(lin_mask, vv2 * lv2, 0.0))
            rs = plsc.cumsum(r)
            plsc.store_scatter(out_v, [jnp.full((dim,), b, jnp.int32)], rs,
                               mask=last_lane)
            return 0

        lax.fori_loop(0, _CHUNK_ROWS, row_body, 0, unroll=False)
        pltpu.sync_copy(out_v, out_hbm.at[wid, pl.ds(c * _CHUNK_ROWS,
                                                     _CHUNK_ROWS)])
        return 0

    lax.fori_loop(0, n_chunks, chunk_body, 0, unroll=False)


def kernel(feature_idx, feature_value, W_linear, bias, W_embed):
    batch, fields = feature_idx.shape
    dim = W_embed.shape[1]
    assert dim == 16 and 16 < fields <= 32
    assert batch % (_NW * _CHUNK_ROWS) == 0
    rows_per_w = batch // _NW
    n_chunks = rows_per_w // _CHUNK_ROWS
    g = _CHUNK_ROWS * fields
    assert g % _GW == 0
    n_groups = g // _GW

    idx_r = feature_idx.reshape(_NW, n_chunks, n_groups, _GW)
    val_r = feature_value.reshape(_NW, n_chunks, g)
    wlin = W_linear.reshape(-1)

    mesh = plsc.VectorSubcoreMesh(core_axis_name="c", subcore_axis_name="s")
    body = functools.partial(_fm_body, n_chunks=n_chunks, n_groups=n_groups,
                             fields=fields, dim=dim)
    out = pl.kernel(
        body,
        out_type=jax.ShapeDtypeStruct((_NW, rows_per_w), jnp.float32),
        mesh=mesh,
        compiler_params=pltpu.CompilerParams(use_tc_tiling_on_sc=False,
                                             needs_layout_passes=False),
        scratch_types=[
            pltpu.VMEM((n_groups, _GW), jnp.int32),
            pltpu.VMEM((g + dim,), jnp.float32),
            pltpu.VMEM((g, dim), jnp.float32),
            pltpu.VMEM((g + dim,), jnp.float32),
            pltpu.VMEM((_CHUNK_ROWS,), jnp.float32),
            pltpu.SemaphoreType.DMA,
        ],
    )(idx_r, val_r, wlin, W_embed)
    return out.reshape(batch, 1) + bias[None, :]


# speedup vs baseline: 1.5143x; 1.1699x over previous
import functools

import jax
import jax.numpy as jnp
from jax import lax
from jax.experimental import pallas as pl
from jax.experimental.pallas import tpu as pltpu
from jax.experimental.pallas import tpu_sc as plsc

_NC = 2
_NS = 16
_NW = _NC * _NS

_CHUNK_ROWS = 64
_GW = 128


def _fm_body(idx_hbm, val_hbm, wlin_hbm, wemb_hbm, out_hbm,
             idx_v, val_v, emb_v, lin_v, out_v, sem,
             *, n_chunks, n_groups, fields, dim):
    wid = lax.axis_index("s") * _NC + lax.axis_index("c")
    g = _CHUNK_ROWS * fields

    val_v[pl.ds(g, dim)] = jnp.zeros((dim,), jnp.float32)
    lin_v[pl.ds(g, dim)] = jnp.zeros((dim,), jnp.float32)

    def chunk_body(c, _):
        pltpu.sync_copy(idx_hbm.at[wid, c], idx_v)
        pltpu.sync_copy(val_hbm.at[wid, c], val_v.at[pl.ds(0, g)])
        copies = []
        for j in range(n_groups):
            copies.append(pltpu.async_copy(
                wemb_hbm.at[idx_v.at[j]],
                emb_v.at[pl.ds(j * _GW, _GW)], sem))
            copies.append(pltpu.async_copy(
                wlin_hbm.at[idx_v.at[j]],
                lin_v.at[pl.ds(j * _GW, _GW)], sem))
        for cp in copies:
            cp.wait()

        def row_body(b, _):
            lanes = lax.iota(jnp.int32, dim)
            last_lane = lanes == (dim - 1)
            lin_mask = lanes < (fields - 16)
            n0 = b * fields
            vv1 = val_v[pl.ds(n0, dim)]
            vv2 = val_v[pl.ds(n0 + dim, dim)]
            lv1 = lin_v[pl.ds(n0, dim)]
            lv2 = lin_v[pl.ds(n0 + dim, dim)]
            acc = jnp.zeros((dim,), jnp.float32)
            accq = jnp.zeros((dim,), jnp.float32)
            for f in range(fields):
                e = emb_v[n0 + f, :]
                v = vv1[f] if f < dim else vv2[f - dim]
                ev = e * v
                acc = acc + ev
                accq = accq + ev * ev
            r = (0.5 * (acc * acc - accq) + vv1 * lv1
                 + jnp.where(lin_mask, vv2 * lv2, 0.0))
            rs = plsc.cumsum(r)
            plsc.store_scatter(out_v, [jnp.full((dim,), b, jnp.int32)], rs,
                               mask=last_lane)
            return 0

        lax.fori_loop(0, _CHUNK_ROWS, row_body, 0, unroll=False)
        pltpu.sync_copy(out_v, out_hbm.at[wid, pl.ds(c * _CHUNK_ROWS,
                                                     _CHUNK_ROWS)])
        return 0

    lax.fori_loop(0, n_chunks, chunk_body, 0, unroll=False)


_TR_COLS = 8192


def _transpose_block(wt_ref, out_ref, xt_ref):
    dim = wt_ref.shape[0]
    xt_ref[...] = wt_ref[...].T
    for k in range(8):
        out_ref[:, k * dim:(k + 1) * dim] = (
            xt_ref[pl.Slice(k, _TR_COLS // 8, 8), :])


def _to_row_major(W_embed):
    n, dim = W_embed.shape
    wt = W_embed.T
    grid = (n + _TR_COLS - 1) // _TR_COLS
    out = pl.pallas_call(
        _transpose_block,
        grid=(grid,),
        in_specs=[pl.BlockSpec((dim, _TR_COLS), lambda i: (0, i))],
        out_specs=pl.BlockSpec((_TR_COLS // 8, 8 * dim), lambda i: (i, 0)),
        out_shape=jax.ShapeDtypeStruct((n // 8, 8 * dim), jnp.float32),
        scratch_shapes=[pltpu.VMEM((_TR_COLS, dim), jnp.float32)],
    )(wt)
    return out.reshape(n, dim)


def kernel(feature_idx, feature_value, W_linear, bias, W_embed):
    batch, fields = feature_idx.shape
    dim = W_embed.shape[1]
    assert dim == 16 and 16 < fields <= 32
    assert batch % (_NW * _CHUNK_ROWS) == 0
    rows_per_w = batch // _NW
    n_chunks = rows_per_w // _CHUNK_ROWS
    g = _CHUNK_ROWS * fields
    assert g % _GW == 0
    n_groups = g // _GW

    idx_r = feature_idx.reshape(_NW, n_chunks, n_groups, _GW)
    val_r = feature_value.reshape(_NW, n_chunks, g)
    wlin = W_linear.reshape(-1)
    wemb_rm = _to_row_major(W_embed)

    mesh = plsc.VectorSubcoreMesh(core_axis_name="c", subcore_axis_name="s")
    body = functools.partial(_fm_body, n_chunks=n_chunks, n_groups=n_groups,
                             fields=fields, dim=dim)
    out = pl.kernel(
        body,
        out_type=jax.ShapeDtypeStruct((_NW, rows_per_w), jnp.float32),
        mesh=mesh,
        compiler_params=pltpu.CompilerParams(use_tc_tiling_on_sc=False,
                                             needs_layout_passes=False),
        scratch_types=[
            pltpu.VMEM((n_groups, _GW), jnp.int32),
            pltpu.VMEM((g + dim,), jnp.float32),
            pltpu.VMEM((g, dim), jnp.float32),
            pltpu.VMEM((g + dim,), jnp.float32),
            pltpu.VMEM((_CHUNK_ROWS,), jnp.float32),
            pltpu.SemaphoreType.DMA,
        ],
    )(idx_r, val_r, wlin, wemb_rm)
    return out.reshape(batch, 1) + bias[None, :]


# speedup vs baseline: 1.7968x; 1.1866x over previous
import functools

import jax
import jax.numpy as jnp
from jax import lax
from jax.experimental import pallas as pl
from jax.experimental.pallas import tpu as pltpu
from jax.experimental.pallas import tpu_sc as plsc

_NC = 2
_NS = 16
_NW = _NC * _NS

_CHUNK_ROWS = 64
_GW = 128


def _fm_body(idx_hbm, idxl_hbm, val_hbm, wlin_hbm, wemb_hbm, out_hbm,
             idx_v, idxl_v, val_v, emb_v, lin_v, out_v, sem,
             *, n_chunks, n_groups, fields, dim):
    wid = lax.axis_index("s") * _NC + lax.axis_index("c")
    g = _CHUNK_ROWS * fields

    val_v[pl.ds(g, dim)] = jnp.zeros((dim,), jnp.float32)
    lin_v[pl.ds(g, dim)] = jnp.zeros((dim,), jnp.float32)

    def chunk_body(c, _):
        pltpu.sync_copy(idx_hbm.at[wid, c], idx_v)
        pltpu.sync_copy(idxl_hbm.at[wid, c], idxl_v)
        pltpu.sync_copy(val_hbm.at[wid, c], val_v.at[pl.ds(0, g)])
        copies = []
        for j in range(n_groups):
            copies.append(pltpu.async_copy(
                wemb_hbm.at[idx_v.at[j]],
                emb_v.at[pl.ds(j * _GW, _GW)], sem))
            copies.append(pltpu.async_copy(
                wlin_hbm.at[idxl_v.at[j]],
                lin_v.at[pl.ds(j * _GW, _GW)], sem))
        for cp in copies:
            cp.wait()

        def row_body(b, _):
            lanes = lax.iota(jnp.int32, dim)
            last_lane = lanes == (dim - 1)
            lin_mask = lanes < (fields - 16)
            n0 = b * fields
            vv1 = val_v[pl.ds(n0, dim)]
            vv2 = val_v[pl.ds(n0 + dim, dim)]
            lv1 = lin_v[pl.ds(n0, dim)]
            lv2 = lin_v[pl.ds(n0 + dim, dim)]
            acc = jnp.zeros((dim,), jnp.float32)
            accq = jnp.zeros((dim,), jnp.float32)
            for f in range(fields):
                e = emb_v[n0 + f, :]
                v = vv1[f] if f < dim else vv2[f - dim]
                ev = e * v
                acc = acc + ev
                accq = accq + ev * ev
            r = (0.5 * (acc * acc - accq) + vv1 * lv1
                 + jnp.where(lin_mask, vv2 * lv2, 0.0))
            rs = plsc.cumsum(r)
            plsc.store_scatter(out_v, [jnp.full((dim,), b, jnp.int32)], rs,
                               mask=last_lane)
            return 0

        lax.fori_loop(0, _CHUNK_ROWS, row_body, 0, unroll=False)
        pltpu.sync_copy(out_v, out_hbm.at[wid, pl.ds(c * _CHUNK_ROWS,
                                                     _CHUNK_ROWS)])
        return 0

    lax.fori_loop(0, n_chunks, chunk_body, 0, unroll=False)


_TR_LINES = 1024


def _perm_matrix(dim):
    import numpy as np
    m = np.zeros((8 * dim, 8 * dim), np.float32)
    for d in range(dim):
        for k in range(8):
            m[d * 8 + k, k * dim + d] = 1.0
    return jnp.asarray(m)


def _transpose_block(v_ref, m_ref, out_ref):
    out_ref[...] = jax.lax.dot_general(
        v_ref[...], m_ref[...], (((0,), (0,)), ((), ())),
        preferred_element_type=jnp.float32)


def _to_row_major(W_embed):
    n, dim = W_embed.shape
    n_phase = n // 8
    v = W_embed.T.reshape(8 * dim, n_phase)
    grid = (n_phase + _TR_LINES - 1) // _TR_LINES
    out = pl.pallas_call(
        _transpose_block,
        grid=(grid,),
        in_specs=[pl.BlockSpec((8 * dim, _TR_LINES), lambda i: (0, i)),
                  pl.BlockSpec((8 * dim, 8 * dim), lambda i: (0, 0))],
        out_specs=pl.BlockSpec((_TR_LINES, 8 * dim), lambda i: (i, 0)),
        out_shape=jax.ShapeDtypeStruct((n_phase, 8 * dim), jnp.float32),
    )(v, _perm_matrix(dim))
    return out.reshape(n, dim)


def kernel(feature_idx, feature_value, W_linear, bias, W_embed):
    batch, fields = feature_idx.shape
    dim = W_embed.shape[1]
    assert dim == 16 and 16 < fields <= 32
    assert batch % (_NW * _CHUNK_ROWS) == 0
    rows_per_w = batch // _NW
    n_chunks = rows_per_w // _CHUNK_ROWS
    g = _CHUNK_ROWS * fields
    assert g % _GW == 0
    n_groups = g // _GW

    n_phase = W_embed.shape[0] // 8
    ridx = 8 * (feature_idx % n_phase) + feature_idx // n_phase
    idx_r = ridx.reshape(_NW, n_chunks, n_groups, _GW)
    idxl_r = feature_idx.reshape(_NW, n_chunks, n_groups, _GW)
    val_r = feature_value.reshape(_NW, n_chunks, g)
    wlin = W_linear.reshape(-1)
    wemb_rm = _to_row_major(W_embed)

    mesh = plsc.VectorSubcoreMesh(core_axis_name="c", subcore_axis_name="s")
    body = functools.partial(_fm_body, n_chunks=n_chunks, n_groups=n_groups,
                             fields=fields, dim=dim)
    out = pl.kernel(
        body,
        out_type=jax.ShapeDtypeStruct((_NW, rows_per_w), jnp.float32),
        mesh=mesh,
        compiler_params=pltpu.CompilerParams(use_tc_tiling_on_sc=False,
                                             needs_layout_passes=False),
        scratch_types=[
            pltpu.VMEM((n_groups, _GW), jnp.int32),
            pltpu.VMEM((n_groups, _GW), jnp.int32),
            pltpu.VMEM((g + dim,), jnp.float32),
            pltpu.VMEM((g, dim), jnp.float32),
            pltpu.VMEM((g + dim,), jnp.float32),
            pltpu.VMEM((_CHUNK_ROWS,), jnp.float32),
            pltpu.SemaphoreType.DMA,
        ],
    )(idx_r, idxl_r, val_r, wlin, wemb_rm)
    return out.reshape(batch, 1) + bias[None, :]


# speedup vs baseline: 2.0707x; 1.1524x over previous
import functools

import jax
import jax.numpy as jnp
from jax import lax
from jax.experimental import pallas as pl
from jax.experimental.pallas import tpu as pltpu
from jax.experimental.pallas import tpu_sc as plsc

_NC = 2
_NS = 16
_NW = _NC * _NS

_CHUNK_ROWS = 64
_GW = 128


def _fm_body(idx_hbm, idxl_hbm, val_hbm, wlin_hbm, wemb_hbm, out_hbm,
             idx_v, idxl_v, val_v, emb_v, lin_v, out_v, sem,
             *, n_chunks, n_groups, fields, dim):
    wid = lax.axis_index("s") * _NC + lax.axis_index("c")
    g = _CHUNK_ROWS * fields

    val_v[pl.ds(g, dim)] = jnp.zeros((dim,), jnp.float32)
    lin_v[pl.ds(g, dim)] = jnp.zeros((dim,), jnp.float32)

    def chunk_body(c, _):
        pltpu.sync_copy(idx_hbm.at[wid, c], idx_v)
        pltpu.sync_copy(idxl_hbm.at[wid, c], idxl_v)
        pltpu.sync_copy(val_hbm.at[wid, c], val_v.at[pl.ds(0, g)])
        copies = []
        for j in range(n_groups):
            copies.append(pltpu.async_copy(
                wemb_hbm.at[idx_v.at[j]],
                emb_v.at[pl.ds(j * _GW, _GW)], sem))
            copies.append(pltpu.async_copy(
                wlin_hbm.at[idxl_v.at[j]],
                lin_v.at[pl.ds(j * _GW, _GW)], sem))
        for cp in copies:
            cp.wait()

        def row_body(b, _):
            lanes = lax.iota(jnp.int32, dim)
            last_lane = lanes == (dim - 1)
            lin_mask = lanes < (fields - 16)
            n0 = b * fields
            vv1 = val_v[pl.ds(n0, dim)]
            vv2 = val_v[pl.ds(n0 + dim, dim)]
            lv1 = lin_v[pl.ds(n0, dim)]
            lv2 = lin_v[pl.ds(n0 + dim, dim)]
            acc = jnp.zeros((dim,), jnp.float32)
            accq = jnp.zeros((dim,), jnp.float32)
            for f in range(fields):
                e = emb_v[n0 + f, :]
                v = vv1[f] if f < dim else vv2[f - dim]
                ev = e * v
                acc = acc + ev
                accq = accq + ev * ev
            r = (0.5 * (acc * acc - accq) + vv1 * lv1
                 + jnp.where(lin_mask, vv2 * lv2, 0.0))
            rs = plsc.cumsum(r)
            plsc.store_scatter(out_v, [jnp.full((dim,), b, jnp.int32)], rs,
                               mask=last_lane)
            return 0

        lax.fori_loop(0, _CHUNK_ROWS, row_body, 0, unroll=False)
        pltpu.sync_copy(out_v, out_hbm.at[wid, pl.ds(c * _CHUNK_ROWS,
                                                     _CHUNK_ROWS)])
        return 0

    lax.fori_loop(0, n_chunks, chunk_body, 0, unroll=False)


_TR_LINES = 1024


_TR_S = 2048


def _repack_block(wt_ref, eye_ref, out_ref):
    dim = wt_ref.shape[0]
    acc = None
    for k in range(8):
        xk = wt_ref[:, k * _TR_S:(k + 1) * _TR_S]
        ek = eye_ref[k * dim:(k + 1) * dim, :]
        p = jax.lax.dot_general(xk, ek, (((0,), (0,)), ((), ())),
                                preferred_element_type=jnp.float32)
        acc = p if acc is None else acc + p
    out_ref[...] = acc


def _to_row_major(W_embed):
    n, dim = W_embed.shape
    wt = W_embed.T
    bs = 8 * _TR_S
    grid = (n + bs - 1) // bs
    out = pl.pallas_call(
        _repack_block,
        grid=(grid,),
        in_specs=[pl.BlockSpec((dim, bs), lambda i: (0, i)),
                  pl.BlockSpec((8 * dim, 8 * dim), lambda i: (0, 0))],
        out_specs=pl.BlockSpec((_TR_S, 8 * dim), lambda i: (i, 0)),
        out_shape=jax.ShapeDtypeStruct((grid * _TR_S, 8 * dim),
                                       jnp.float32),
    )(wt, jnp.eye(8 * dim, dtype=jnp.float32))
    return out.reshape(grid * bs, dim)


def kernel(feature_idx, feature_value, W_linear, bias, W_embed):
    batch, fields = feature_idx.shape
    dim = W_embed.shape[1]
    assert dim == 16 and 16 < fields <= 32
    assert batch % (_NW * _CHUNK_ROWS) == 0
    rows_per_w = batch // _NW
    n_chunks = rows_per_w // _CHUNK_ROWS
    g = _CHUNK_ROWS * fields
    assert g % _GW == 0
    n_groups = g // _GW

    bs = 8 * _TR_S
    r = feature_idx
    ridx = bs * (r // bs) + 8 * (r % _TR_S) + (r % bs) // _TR_S
    idx_r = ridx.reshape(_NW, n_chunks, n_groups, _GW)
    idxl_r = feature_idx.reshape(_NW, n_chunks, n_groups, _GW)
    val_r = feature_value.reshape(_NW, n_chunks, g)
    wlin = W_linear.reshape(-1)
    wemb_rm = _to_row_major(W_embed)

    mesh = plsc.VectorSubcoreMesh(core_axis_name="c", subcore_axis_name="s")
    body = functools.partial(_fm_body, n_chunks=n_chunks, n_groups=n_groups,
                             fields=fields, dim=dim)
    out = pl.kernel(
        body,
        out_type=jax.ShapeDtypeStruct((_NW, rows_per_w), jnp.float32),
        mesh=mesh,
        compiler_params=pltpu.CompilerParams(use_tc_tiling_on_sc=False,
                                             needs_layout_passes=False),
        scratch_types=[
            pltpu.VMEM((n_groups, _GW), jnp.int32),
            pltpu.VMEM((n_groups, _GW), jnp.int32),
            pltpu.VMEM((g + dim,), jnp.float32),
            pltpu.VMEM((g, dim), jnp.float32),
            pltpu.VMEM((g + dim,), jnp.float32),
            pltpu.VMEM((_CHUNK_ROWS,), jnp.float32),
            pltpu.SemaphoreType.DMA,
        ],
    )(idx_r, idxl_r, val_r, wlin, wemb_rm)
    return out.reshape(batch, 1) + bias[None, :]


# speedup vs baseline: 3.2535x; 1.5712x over previous
import functools

import jax
import jax.numpy as jnp
from jax import lax
from jax.experimental import pallas as pl
from jax.experimental.pallas import tpu as pltpu
from jax.experimental.pallas import tpu_sc as plsc

_NC = 2
_NS = 16
_NW = _NC * _NS

_CHUNK_ROWS = 64
_GW = 128


def _fm_body(idx_hbm, idxl_hbm, val_hbm, wlin_hbm, wemb_hbm, out_hbm,
             idx_v, idxl_v, val_v, emb_v, lin_v, out_v, sem,
             *, n_chunks, n_groups, fields, dim):
    wid = lax.axis_index("s") * _NC + lax.axis_index("c")
    g = _CHUNK_ROWS * fields

    val_v[pl.ds(g, dim)] = jnp.zeros((dim,), jnp.float32)
    lin_v[pl.ds(g, dim)] = jnp.zeros((dim,), jnp.float32)

    def chunk_body(c, _):
        pltpu.sync_copy(idx_hbm.at[wid, c], idx_v)
        pltpu.sync_copy(idxl_hbm.at[wid, c], idxl_v)
        pltpu.sync_copy(val_hbm.at[wid, c], val_v.at[pl.ds(0, g)])
        copies = []
        for j in range(n_groups):
            copies.append(pltpu.async_copy(
                wemb_hbm.at[idx_v.at[j]],
                emb_v.at[pl.ds(j * _GW, _GW)], sem))
            copies.append(pltpu.async_copy(
                wlin_hbm.at[idxl_v.at[j]],
                lin_v.at[pl.ds(j * _GW, _GW)], sem))
        for cp in copies:
            cp.wait()

        def row_body(b, _):
            lanes = lax.iota(jnp.int32, dim)
            last_lane = lanes == (dim - 1)
            lin_mask = lanes < (fields - 16)
            n0 = b * fields
            vv1 = val_v[pl.ds(n0, dim)]
            vv2 = val_v[pl.ds(n0 + dim, dim)]
            lv1 = lin_v[pl.ds(n0, dim)]
            lv2 = lin_v[pl.ds(n0 + dim, dim)]
            acc = jnp.zeros((dim,), jnp.float32)
            accq = jnp.zeros((dim,), jnp.float32)
            for f in range(fields):
                e = emb_v[n0 + f, :]
                v = vv1[f] if f < dim else vv2[f - dim]
                ev = e * v
                acc = acc + ev
                accq = accq + ev * ev
            r = (0.5 * (acc * acc - accq) + vv1 * lv1
                 + jnp.where(lin_mask, vv2 * lv2, 0.0))
            rs = plsc.cumsum(r)
            plsc.store_scatter(out_v, [jnp.full((dim,), b, jnp.int32)], rs,
                               mask=last_lane)
            return 0

        lax.fori_loop(0, _CHUNK_ROWS, row_body, 0, unroll=False)
        pltpu.sync_copy(out_v, out_hbm.at[wid, pl.ds(c * _CHUNK_ROWS,
                                                     _CHUNK_ROWS)])
        return 0

    lax.fori_loop(0, n_chunks, chunk_body, 0, unroll=False)


_TR_S = 2048


def _repack_block(wt_ref, wl_ref, out_ref, out_lin_ref):
    x = wt_ref[...]
    xcat = jnp.concatenate(
        [x[:, k * _TR_S:(k + 1) * _TR_S] for k in range(8)], axis=0)
    out_ref[...] = xcat.T
    out_lin_ref[...] = wl_ref[...].reshape(out_lin_ref.shape)


def _repack(W_embed, W_linear):
    n, dim = W_embed.shape
    wt = W_embed.T
    wl = W_linear.T
    bs = 8 * _TR_S
    grid = (n + bs - 1) // bs
    lpb = bs // (8 * dim)
    out, out_lin = pl.pallas_call(
        _repack_block,
        grid=(grid,),
        in_specs=[pl.BlockSpec((dim, bs), lambda i: (0, i)),
                  pl.BlockSpec((1, bs), lambda i: (0, i))],
        out_specs=[pl.BlockSpec((_TR_S, 8 * dim), lambda i: (i, 0)),
                   pl.BlockSpec((lpb, 8 * dim), lambda i: (i, 0))],
        out_shape=[jax.ShapeDtypeStruct((grid * _TR_S, 8 * dim),
                                        jnp.float32),
                   jax.ShapeDtypeStruct((grid * lpb, 8 * dim),
                                        jnp.float32)],
    )(wt, wl)
    return out.reshape(grid * bs, dim), out_lin.reshape(-1)


def kernel(feature_idx, feature_value, W_linear, bias, W_embed):
    batch, fields = feature_idx.shape
    dim = W_embed.shape[1]
    assert dim == 16 and 16 < fields <= 32
    assert batch % (_NW * _CHUNK_ROWS) == 0
    rows_per_w = batch // _NW
    n_chunks = rows_per_w // _CHUNK_ROWS
    g = _CHUNK_ROWS * fields
    assert g % _GW == 0
    n_groups = g // _GW

    bs = 8 * _TR_S
    r = feature_idx
    ridx = bs * (r // bs) + 8 * (r % _TR_S) + (r % bs) // _TR_S
    idx_r = ridx.reshape(_NW, n_chunks, n_groups, _GW)
    idxl_r = feature_idx.reshape(_NW, n_chunks, n_groups, _GW)
    val_r = feature_value.reshape(_NW, n_chunks, g)
    wemb_rm, wlin = _repack(W_embed, W_linear)

    mesh = plsc.VectorSubcoreMesh(core_axis_name="c", subcore_axis_name="s")
    body = functools.partial(_fm_body, n_chunks=n_chunks, n_groups=n_groups,
                             fields=fields, dim=dim)
    out = pl.kernel(
        body,
        out_type=jax.ShapeDtypeStruct((_NW, rows_per_w), jnp.float32),
        mesh=mesh,
        compiler_params=pltpu.CompilerParams(use_tc_tiling_on_sc=False,
                                             needs_layout_passes=False),
        scratch_types=[
            pltpu.VMEM((n_groups, _GW), jnp.int32),
            pltpu.VMEM((n_groups, _GW), jnp.int32),
            pltpu.VMEM((g + dim,), jnp.float32),
            pltpu.VMEM((g, dim), jnp.float32),
            pltpu.VMEM((g + dim,), jnp.float32),
            pltpu.VMEM((_CHUNK_ROWS,), jnp.float32),
            pltpu.SemaphoreType.DMA,
        ],
    )(idx_r, idxl_r, val_r, wlin, wemb_rm)
    return out.reshape(batch, 1) + bias[None, :]


# speedup vs baseline: 3.5751x; 1.0989x over previous
import functools

import jax
import jax.numpy as jnp
from jax import lax
from jax.experimental import pallas as pl
from jax.experimental.pallas import tpu as pltpu
from jax.experimental.pallas import tpu_sc as plsc

_NC = 2
_NS = 16
_NW = _NC * _NS

_CHUNK_ROWS = 64
_GW = 128


def _fm_body(idx_hbm, idxl_hbm, val_hbm, wlin_hbm, wemb_hbm, out_hbm,
             idx_v, idxl_v, val_v, emb_v, lin_v, out_v, sems,
             *, n_chunks, n_groups, fields, dim):
    wid = lax.axis_index("s") * _NC + lax.axis_index("c")
    g = _CHUNK_ROWS * fields

    for buf in range(2):
        val_v[buf, pl.ds(g, dim)] = jnp.zeros((dim,), jnp.float32)
        lin_v[buf, pl.ds(g, dim)] = jnp.zeros((dim,), jnp.float32)

    def start_chunk(c, buf):
        pltpu.sync_copy(idx_hbm.at[wid, c], idx_v.at[buf])
        pltpu.sync_copy(idxl_hbm.at[wid, c], idxl_v.at[buf])
        pltpu.sync_copy(val_hbm.at[wid, c], val_v.at[buf, pl.ds(0, g)])
        copies = []
        for j in range(n_groups):
            copies.append(pltpu.async_copy(
                wemb_hbm.at[idx_v.at[buf, j]],
                emb_v.at[buf, pl.ds(j * _GW, _GW)], sems.at[buf]))
            copies.append(pltpu.async_copy(
                wlin_hbm.at[idxl_v.at[buf, j]],
                lin_v.at[buf, pl.ds(j * _GW, _GW)], sems.at[buf]))
        return copies

    def compute_chunk(c, buf):
        def row_body(b, _):
            lanes = lax.iota(jnp.int32, dim)
            last_lane = lanes == (dim - 1)
            lin_mask = lanes < (fields - 16)
            n0 = b * fields
            vv1 = val_v[buf, pl.ds(n0, dim)]
            vv2 = val_v[buf, pl.ds(n0 + dim, dim)]
            lv1 = lin_v[buf, pl.ds(n0, dim)]
            lv2 = lin_v[buf, pl.ds(n0 + dim, dim)]
            acc = jnp.zeros((dim,), jnp.float32)
            accq = jnp.zeros((dim,), jnp.float32)
            for f in range(fields):
                e = emb_v[buf, n0 + f, :]
                v = vv1[f] if f < dim else vv2[f - dim]
                ev = e * v
                acc = acc + ev
                accq = accq + ev * ev
            r = (0.5 * (acc * acc - accq) + vv1 * lv1
                 + jnp.where(lin_mask, vv2 * lv2, 0.0))
            rs = plsc.cumsum(r)
            plsc.store_scatter(out_v, [jnp.full((dim,), b, jnp.int32)], rs,
                               mask=last_lane)
            return 0

        lax.fori_loop(0, _CHUNK_ROWS, row_body, 0, unroll=False)
        pltpu.sync_copy(out_v, out_hbm.at[wid, pl.ds(c * _CHUNK_ROWS,
                                                     _CHUNK_ROWS)])

    copies = start_chunk(0, 0)
    for c in range(n_chunks):
        buf = c % 2
        for cp in copies:
            cp.wait()
        if c + 1 < n_chunks:
            copies = start_chunk(c + 1, 1 - buf)
        compute_chunk(c, buf)


_TR_S = 2048


def _repack_block(wt_ref, wl_ref, out_ref, out_lin_ref):
    x = wt_ref[...]
    xcat = jnp.concatenate(
        [x[:, k * _TR_S:(k + 1) * _TR_S] for k in range(8)], axis=0)
    out_ref[...] = xcat.T
    out_lin_ref[...] = wl_ref[...].reshape(out_lin_ref.shape)


def _repack(W_embed, W_linear):
    n, dim = W_embed.shape
    wt = W_embed.T
    wl = W_linear.T
    bs = 8 * _TR_S
    grid = (n + bs - 1) // bs
    lpb = bs // (8 * dim)
    out, out_lin = pl.pallas_call(
        _repack_block,
        grid=(grid,),
        in_specs=[pl.BlockSpec((dim, bs), lambda i: (0, i)),
                  pl.BlockSpec((1, bs), lambda i: (0, i))],
        out_specs=[pl.BlockSpec((_TR_S, 8 * dim), lambda i: (i, 0)),
                   pl.BlockSpec((lpb, 8 * dim), lambda i: (i, 0))],
        out_shape=[jax.ShapeDtypeStruct((grid * _TR_S, 8 * dim),
                                        jnp.float32),
                   jax.ShapeDtypeStruct((grid * lpb, 8 * dim),
                                        jnp.float32)],
    )(wt, wl)
    return out.reshape(grid * bs, dim), out_lin.reshape(-1)


def kernel(feature_idx, feature_value, W_linear, bias, W_embed):
    batch, fields = feature_idx.shape
    dim = W_embed.shape[1]
    assert dim == 16 and 16 < fields <= 32
    assert batch % (_NW * _CHUNK_ROWS) == 0
    rows_per_w = batch // _NW
    n_chunks = rows_per_w // _CHUNK_ROWS
    g = _CHUNK_ROWS * fields
    assert g % _GW == 0
    n_groups = g // _GW

    bs = 8 * _TR_S
    r = feature_idx
    ridx = bs * (r // bs) + 8 * (r % _TR_S) + (r % bs) // _TR_S
    idx_r = ridx.reshape(_NW, n_chunks, n_groups, _GW)
    idxl_r = feature_idx.reshape(_NW, n_chunks, n_groups, _GW)
    val_r = feature_value.reshape(_NW, n_chunks, g)
    wemb_rm, wlin = _repack(W_embed, W_linear)

    mesh = plsc.VectorSubcoreMesh(core_axis_name="c", subcore_axis_name="s")
    body = functools.partial(_fm_body, n_chunks=n_chunks, n_groups=n_groups,
                             fields=fields, dim=dim)
    out = pl.kernel(
        body,
        out_type=jax.ShapeDtypeStruct((_NW, rows_per_w), jnp.float32),
        mesh=mesh,
        compiler_params=pltpu.CompilerParams(use_tc_tiling_on_sc=False,
                                             needs_layout_passes=False),
        scratch_types=[
            pltpu.VMEM((2, n_groups, _GW), jnp.int32),
            pltpu.VMEM((2, n_groups, _GW), jnp.int32),
            pltpu.VMEM((2, g + dim), jnp.float32),
            pltpu.VMEM((2, g, dim), jnp.float32),
            pltpu.VMEM((2, g + dim), jnp.float32),
            pltpu.VMEM((_CHUNK_ROWS,), jnp.float32),
            pltpu.SemaphoreType.DMA((2,)),
        ],
    )(idx_r, idxl_r, val_r, wlin, wemb_rm)
    return out.reshape(batch, 1) + bias[None, :]


# speedup vs baseline: 4.0325x; 1.1279x over previous
import functools

import jax
import jax.numpy as jnp
from jax import lax
from jax.experimental import pallas as pl
from jax.experimental.pallas import tpu as pltpu
from jax.experimental.pallas import tpu_sc as plsc

_NC = 2
_NS = 16
_NW = _NC * _NS

_CHUNK_ROWS = 64
_GW = 128


def _fm_body(idx_hbm, val_hbm, wlin_hbm, wemb_hbm, out_hbm,
             idx_v, idx2_v, val_v, emb_v, lin_v, out_v, sems,
             *, n_chunks, n_groups, fields, dim):
    wid = lax.axis_index("s") * _NC + lax.axis_index("c")
    g = _CHUNK_ROWS * fields

    for buf in range(2):
        val_v[buf, pl.ds(g, dim)] = jnp.zeros((dim,), jnp.float32)
        lin_v[buf, pl.ds(g, dim)] = jnp.zeros((dim,), jnp.float32)

    def start_chunk(c, buf):
        pltpu.sync_copy(idx_hbm.at[wid, c], idx_v.at[buf])
        pltpu.sync_copy(val_hbm.at[wid, c], val_v.at[buf, pl.ds(0, g)])
        copies = []
        for j in range(n_groups):
            copies.append(pltpu.async_copy(
                wlin_hbm.at[idx_v.at[buf, j]],
                lin_v.at[buf, pl.ds(j * _GW, _GW)], sems.at[buf]))

        def remap_body(t, _):
            jj = t >> 3
            off = (t & 7) * dim
            x = idx_v[buf, jj, pl.ds(off, dim)]
            y = ((x & -(8 * _TR_S)) + 8 * (x & (_TR_S - 1))
                 + ((x & (8 * _TR_S - 1)) >> jnp.int32(_TR_S.bit_length()
                                                      - 1)))
            idx2_v[buf, jj, pl.ds(off, dim)] = y
            return 0

        lax.fori_loop(0, n_groups * (_GW // dim), remap_body, 0,
                      unroll=False)
        for j in range(n_groups):
            copies.append(pltpu.async_copy(
                wemb_hbm.at[idx2_v.at[buf, j]],
                emb_v.at[buf, pl.ds(j * _GW, _GW)], sems.at[buf]))
        return copies

    def compute_chunk(c, buf):
        def row_body(b, _):
            lanes = lax.iota(jnp.int32, dim)
            last_lane = lanes == (dim - 1)
            lin_mask = lanes < (fields - 16)
            n0 = b * fields
            vv1 = val_v[buf, pl.ds(n0, dim)]
            vv2 = val_v[buf, pl.ds(n0 + dim, dim)]
            lv1 = lin_v[buf, pl.ds(n0, dim)]
            lv2 = lin_v[buf, pl.ds(n0 + dim, dim)]
            acc = jnp.zeros((dim,), jnp.float32)
            accq = jnp.zeros((dim,), jnp.float32)
            for f in range(fields):
                e = emb_v[buf, n0 + f, :]
                v = vv1[f] if f < dim else vv2[f - dim]
                ev = e * v
                acc = acc + ev
                accq = accq + ev * ev
            r = (0.5 * (acc * acc - accq) + vv1 * lv1
                 + jnp.where(lin_mask, vv2 * lv2, 0.0))
            rs = plsc.cumsum(r)
            plsc.store_scatter(out_v, [jnp.full((dim,), b, jnp.int32)], rs,
                               mask=last_lane)
            return 0

        lax.fori_loop(0, _CHUNK_ROWS, row_body, 0, unroll=False)
        pltpu.sync_copy(out_v, out_hbm.at[wid, pl.ds(c * _CHUNK_ROWS,
                                                     _CHUNK_ROWS)])

    copies = start_chunk(0, 0)
    for c in range(n_chunks):
        buf = c % 2
        for cp in copies:
            cp.wait()
        if c + 1 < n_chunks:
            copies = start_chunk(c + 1, 1 - buf)
        compute_chunk(c, buf)


_TR_S = 2048


def _repack_block(wt_ref, wl_ref, out_ref, out_lin_ref):
    x = wt_ref[...]
    xcat = jnp.concatenate(
        [x[:, k * _TR_S:(k + 1) * _TR_S] for k in range(8)], axis=0)
    out_ref[...] = xcat.T
    out_lin_ref[...] = wl_ref[...].reshape(out_lin_ref.shape)


def _repack(W_embed, W_linear):
    n, dim = W_embed.shape
    wt = W_embed.T
    wl = W_linear.T
    bs = 8 * _TR_S
    grid = (n + bs - 1) // bs
    lpb = bs // (8 * dim)
    out, out_lin = pl.pallas_call(
        _repack_block,
        grid=(grid,),
        in_specs=[pl.BlockSpec((dim, bs), lambda i: (0, i)),
                  pl.BlockSpec((1, bs), lambda i: (0, i))],
        out_specs=[pl.BlockSpec((_TR_S, 8 * dim), lambda i: (i, 0)),
                   pl.BlockSpec((lpb, 8 * dim), lambda i: (i, 0))],
        out_shape=[jax.ShapeDtypeStruct((grid * _TR_S, 8 * dim),
                                        jnp.float32),
                   jax.ShapeDtypeStruct((grid * lpb, 8 * dim),
                                        jnp.float32)],
    )(wt, wl)
    return out.reshape(grid * bs, dim), out_lin.reshape(-1)


def kernel(feature_idx, feature_value, W_linear, bias, W_embed):
    batch, fields = feature_idx.shape
    dim = W_embed.shape[1]
    assert dim == 16 and 16 < fields <= 32
    assert batch % (_NW * _CHUNK_ROWS) == 0
    rows_per_w = batch // _NW
    n_chunks = rows_per_w // _CHUNK_ROWS
    g = _CHUNK_ROWS * fields
    assert g % _GW == 0
    n_groups = g // _GW

    idx_r = feature_idx.reshape(_NW, n_chunks, n_groups, _GW)
    val_r = feature_value.reshape(_NW, n_chunks, g)
    wemb_rm, wlin = _repack(W_embed, W_linear)

    mesh = plsc.VectorSubcoreMesh(core_axis_name="c", subcore_axis_name="s")
    body = functools.partial(_fm_body, n_chunks=n_chunks, n_groups=n_groups,
                             fields=fields, dim=dim)
    out = pl.kernel(
        body,
        out_type=jax.ShapeDtypeStruct((_NW, rows_per_w), jnp.float32),
        mesh=mesh,
        compiler_params=pltpu.CompilerParams(use_tc_tiling_on_sc=False,
                                             needs_layout_passes=False),
        scratch_types=[
            pltpu.VMEM((2, n_groups, _GW), jnp.int32),
            pltpu.VMEM((2, n_groups, _GW), jnp.int32),
            pltpu.VMEM((2, g + dim), jnp.float32),
            pltpu.VMEM((2, g, dim), jnp.float32),
            pltpu.VMEM((2, g + dim), jnp.float32),
            pltpu.VMEM((_CHUNK_ROWS,), jnp.float32),
            pltpu.SemaphoreType.DMA((2,)),
        ],
    )(idx_r, val_r, wlin, wemb_rm)
    return out.reshape(batch, 1) + bias[None, :]


# speedup vs baseline: 4.5439x; 1.1268x over previous
import functools

import jax
import jax.numpy as jnp
from jax import lax
from jax.experimental import pallas as pl
from jax.experimental.pallas import tpu as pltpu
from jax.experimental.pallas import tpu_sc as plsc

_NC = 2
_NS = 16
_NW = _NC * _NS

_CHUNK_ROWS = 64
_GW = 128


def _fm_body(idx_hbm, val_hbm, wlin_hbm, wemb_hbm, out_hbm,
             idx_v, idx2_v, val_v, emb_v, lin_v, out_v, sems,
             *, n_chunks, n_groups, fields, dim):
    wid = lax.axis_index("s") * _NC + lax.axis_index("c")
    g = _CHUNK_ROWS * fields

    for buf in range(2):
        val_v[buf, pl.ds(g, dim)] = jnp.zeros((dim,), jnp.float32)
        lin_v[buf, pl.ds(g, dim)] = jnp.zeros((dim,), jnp.float32)

    def start_chunk(c, buf):
        pltpu.sync_copy(idx_hbm.at[wid, c], idx_v.at[buf])
        pltpu.sync_copy(val_hbm.at[wid, c], val_v.at[buf, pl.ds(0, g)])
        copies = []
        for j in range(n_groups):
            copies.append(pltpu.async_copy(
                wlin_hbm.at[idx_v.at[buf, j]],
                lin_v.at[buf, pl.ds(j * _GW, _GW)], sems.at[buf]))

        def remap_body(t, _):
            jj = t >> 3
            off = (t & 7) * dim
            x = idx_v[buf, jj, pl.ds(off, dim)]
            y = ((x & -(8 * _TR_S)) + 8 * (x & (_TR_S - 1))
                 + ((x & (8 * _TR_S - 1)) >> jnp.int32(_TR_S.bit_length()
                                                      - 1)))
            idx2_v[buf, jj, pl.ds(off, dim)] = y
            return 0

        lax.fori_loop(0, n_groups * (_GW // dim), remap_body, 0,
                      unroll=False)
        for j in range(n_groups):
            copies.append(pltpu.async_copy(
                wemb_hbm.at[idx2_v.at[buf, j]],
                emb_v.at[buf, pl.ds(j * _GW, _GW)], sems.at[buf]))
        return copies

    def compute_chunk(c, buf):
        def row_body(b, _):
            lanes = lax.iota(jnp.int32, dim)
            last_lane = lanes == (dim - 1)
            lin_mask = lanes < (fields - 16)
            n0 = b * fields
            vv1 = val_v[buf, pl.ds(n0, dim)]
            vv2 = val_v[buf, pl.ds(n0 + dim, dim)]
            lv1 = lin_v[buf, pl.ds(n0, dim)]
            lv2 = lin_v[buf, pl.ds(n0 + dim, dim)]
            acc = jnp.zeros((dim,), jnp.float32)
            accq = jnp.zeros((dim,), jnp.float32)
            for f in range(fields):
                e = emb_v[buf, n0 + f, :]
                v = vv1[f] if f < dim else vv2[f - dim]
                ev = e * v
                acc = acc + ev
                accq = accq + ev * ev
            r = (0.5 * (acc * acc - accq) + vv1 * lv1
                 + jnp.where(lin_mask, vv2 * lv2, 0.0))
            rs = plsc.cumsum(r)
            plsc.store_scatter(out_v, [jnp.full((dim,), b, jnp.int32)], rs,
                               mask=last_lane)
            return 0

        lax.fori_loop(0, _CHUNK_ROWS, row_body, 0, unroll=False)
        pltpu.sync_copy(out_v, out_hbm.at[wid, pl.ds(c * _CHUNK_ROWS,
                                                     _CHUNK_ROWS)])

    copies = start_chunk(0, 0)
    for c in range(n_chunks):
        buf = c % 2
        for cp in copies:
            cp.wait()
        if c + 1 < n_chunks:
            copies = start_chunk(c + 1, 1 - buf)
        compute_chunk(c, buf)


_TR_S = 4096


def _repack_block(wt_ref, wl_ref, out_ref, out_lin_ref):
    x = wt_ref[...]
    xcat = jnp.concatenate(
        [x[:, k * _TR_S:(k + 1) * _TR_S] for k in range(8)], axis=0)
    out_ref[...] = xcat.T
    out_lin_ref[...] = wl_ref[...].reshape(out_lin_ref.shape)


def _repack(W_embed, W_linear):
    n, dim = W_embed.shape
    wt = W_embed.T
    wl = W_linear.T
    bs = 8 * _TR_S
    grid = (n + bs - 1) // bs
    lpb = bs // (8 * dim)
    out, out_lin = pl.pallas_call(
        _repack_block,
        grid=(grid,),
        in_specs=[pl.BlockSpec((dim, bs), lambda i: (0, i)),
                  pl.BlockSpec((1, bs), lambda i: (0, i))],
        out_specs=[pl.BlockSpec((_TR_S, 8 * dim), lambda i: (i, 0)),
                   pl.BlockSpec((lpb, 8 * dim), lambda i: (i, 0))],
        out_shape=[jax.ShapeDtypeStruct((grid * _TR_S, 8 * dim),
                                        jnp.float32),
                   jax.ShapeDtypeStruct((grid * lpb, 8 * dim),
                                        jnp.float32)],
    )(wt, wl)
    return out.reshape(grid * bs, dim), out_lin.reshape(-1)


def kernel(feature_idx, feature_value, W_linear, bias, W_embed):
    batch, fields = feature_idx.shape
    dim = W_embed.shape[1]
    assert dim == 16 and 16 < fields <= 32
    assert batch % (_NW * _CHUNK_ROWS) == 0
    rows_per_w = batch // _NW
    n_chunks = rows_per_w // _CHUNK_ROWS
    g = _CHUNK_ROWS * fields
    assert g % _GW == 0
    n_groups = g // _GW

    idx_r = feature_idx.reshape(_NW, n_chunks, n_groups, _GW)
    val_r = feature_value.reshape(_NW, n_chunks, g)
    wemb_rm, wlin = _repack(W_embed, W_linear)

    mesh = plsc.VectorSubcoreMesh(core_axis_name="c", subcore_axis_name="s")
    body = functools.partial(_fm_body, n_chunks=n_chunks, n_groups=n_groups,
                             fields=fields, dim=dim)
    out = pl.kernel(
        body,
        out_type=jax.ShapeDtypeStruct((_NW, rows_per_w), jnp.float32),
        mesh=mesh,
        compiler_params=pltpu.CompilerParams(use_tc_tiling_on_sc=False,
                                             needs_layout_passes=False),
        scratch_types=[
            pltpu.VMEM((2, n_groups, _GW), jnp.int32),
            pltpu.VMEM((2, n_groups, _GW), jnp.int32),
            pltpu.VMEM((2, g + dim), jnp.float32),
            pltpu.VMEM((2, g, dim), jnp.float32),
            pltpu.VMEM((2, g + dim), jnp.float32),
            pltpu.VMEM((_CHUNK_ROWS,), jnp.float32),
            pltpu.SemaphoreType.DMA((2,)),
        ],
    )(idx_r, val_r, wlin, wemb_rm)
    return out.reshape(batch, 1) + bias[None, :]


# speedup vs baseline: 4.7242x; 1.0397x over previous
import functools

import jax
import jax.numpy as jnp
from jax import lax
from jax.experimental import pallas as pl
from jax.experimental.pallas import tpu as pltpu
from jax.experimental.pallas import tpu_sc as plsc

_NC = 2
_NS = 16
_NW = _NC * _NS

_CHUNK_ROWS = 64
_GW = 128


def _fm_body(idx_hbm, val_hbm, wlin_hbm, wemb_hbm, out_hbm,
             idx_v, idx2_v, val_v, emb_v, lin_v, out_v, sems,
             *, n_chunks, n_groups, fields, dim):
    wid = lax.axis_index("s") * _NC + lax.axis_index("c")
    g = _CHUNK_ROWS * fields

    for buf in range(2):
        val_v[buf, pl.ds(g, dim)] = jnp.zeros((dim,), jnp.float32)
        lin_v[buf, pl.ds(g, dim)] = jnp.zeros((dim,), jnp.float32)

    def start_chunk(c, buf):
        pltpu.sync_copy(idx_hbm.at[wid, c], idx_v.at[buf])
        pltpu.sync_copy(val_hbm.at[wid, c], val_v.at[buf, pl.ds(0, g)])
        copies = []
        for j in range(n_groups):
            copies.append(pltpu.async_copy(
                wlin_hbm.at[idx_v.at[buf, j]],
                lin_v.at[buf, pl.ds(j * _GW, _GW)], sems.at[buf]))

        def remap_body(t, _):
            jj = t >> 3
            off = (t & 7) * dim
            x = idx_v[buf, jj, pl.ds(off, dim)]
            y = ((x & -(8 * _TR_S)) + 8 * (x & (_TR_S - 1))
                 + ((x & (8 * _TR_S - 1)) >> jnp.int32(_TR_S.bit_length()
                                                      - 1)))
            idx2_v[buf, jj, pl.ds(off, dim)] = y
            return 0

        lax.fori_loop(0, n_groups * (_GW // dim), remap_body, 0,
                      unroll=False)
        for j in range(n_groups):
            copies.append(pltpu.async_copy(
                wemb_hbm.at[idx2_v.at[buf, j]],
                emb_v.at[buf, pl.ds(j * _GW, _GW)], sems.at[buf]))
        return copies

    def compute_chunk(c, buf):
        def row_body(b, _):
            lanes = lax.iota(jnp.int32, dim)
            last_lane = lanes == (dim - 1)
            lin_mask = lanes < (fields - 16)
            n0 = b * fields
            vv1 = val_v[buf, pl.ds(n0, dim)]
            vv2 = val_v[buf, pl.ds(n0 + dim, dim)]
            lv1 = lin_v[buf, pl.ds(n0, dim)]
            lv2 = lin_v[buf, pl.ds(n0 + dim, dim)]
            acc = jnp.zeros((dim,), jnp.float32)
            accq = jnp.zeros((dim,), jnp.float32)
            for f in range(fields):
                e = emb_v[buf, n0 + f, :]
                v = vv1[f] if f < dim else vv2[f - dim]
                ev = e * v
                acc = acc + ev
                accq = accq + ev * ev
            r = (0.5 * (acc * acc - accq) + vv1 * lv1
                 + jnp.where(lin_mask, vv2 * lv2, 0.0))
            rs = plsc.cumsum(r)
            plsc.store_scatter(out_v, [jnp.full((dim,), b, jnp.int32)], rs,
                               mask=last_lane)
            return 0

        lax.fori_loop(0, _CHUNK_ROWS, row_body, 0, unroll=False)
        pltpu.sync_copy(out_v, out_hbm.at[wid, pl.ds(c * _CHUNK_ROWS,
                                                     _CHUNK_ROWS)])

    copies = start_chunk(0, 0)
    for c in range(n_chunks):
        buf = c % 2
        for cp in copies:
            cp.wait()
        if c + 1 < n_chunks:
            copies = start_chunk(c + 1, 1 - buf)
        compute_chunk(c, buf)


_TR_S = 8192


def _repack_block(wt_ref, wl_ref, out_ref, out_lin_ref):
    x = wt_ref[...]
    xcat = jnp.concatenate(
        [x[:, k * _TR_S:(k + 1) * _TR_S] for k in range(8)], axis=0)
    out_ref[...] = xcat.T
    out_lin_ref[...] = wl_ref[...].reshape(out_lin_ref.shape)


def _repack(W_embed, W_linear):
    n, dim = W_embed.shape
    wt = W_embed.T
    wl = W_linear.T
    bs = 8 * _TR_S
    grid = (n + bs - 1) // bs
    lpb = bs // (8 * dim)
    out, out_lin = pl.pallas_call(
        _repack_block,
        grid=(grid,),
        in_specs=[pl.BlockSpec((dim, bs), lambda i: (0, i)),
                  pl.BlockSpec((1, bs), lambda i: (0, i))],
        out_specs=[pl.BlockSpec((_TR_S, 8 * dim), lambda i: (i, 0)),
                   pl.BlockSpec((lpb, 8 * dim), lambda i: (i, 0))],
        out_shape=[jax.ShapeDtypeStruct((grid * _TR_S, 8 * dim),
                                        jnp.float32),
                   jax.ShapeDtypeStruct((grid * lpb, 8 * dim),
                                        jnp.float32)],
    )(wt, wl)
    return out.reshape(grid * bs, dim), out_lin.reshape(-1)


def kernel(feature_idx, feature_value, W_linear, bias, W_embed):
    batch, fields = feature_idx.shape
    dim = W_embed.shape[1]
    assert dim == 16 and 16 < fields <= 32
    assert batch % (_NW * _CHUNK_ROWS) == 0
    rows_per_w = batch // _NW
    n_chunks = rows_per_w // _CHUNK_ROWS
    g = _CHUNK_ROWS * fields
    assert g % _GW == 0
    n_groups = g // _GW

    idx_r = feature_idx.reshape(_NW, n_chunks, n_groups, _GW)
    val_r = feature_value.reshape(_NW, n_chunks, g)
    wemb_rm, wlin = _repack(W_embed, W_linear)

    mesh = plsc.VectorSubcoreMesh(core_axis_name="c", subcore_axis_name="s")
    body = functools.partial(_fm_body, n_chunks=n_chunks, n_groups=n_groups,
                             fields=fields, dim=dim)
    out = pl.kernel(
        body,
        out_type=jax.ShapeDtypeStruct((_NW, rows_per_w), jnp.float32),
        mesh=mesh,
        compiler_params=pltpu.CompilerParams(use_tc_tiling_on_sc=False,
                                             needs_layout_passes=False),
        scratch_types=[
            pltpu.VMEM((2, n_groups, _GW), jnp.int32),
            pltpu.VMEM((2, n_groups, _GW), jnp.int32),
            pltpu.VMEM((2, g + dim), jnp.float32),
            pltpu.VMEM((2, g, dim), jnp.float32),
            pltpu.VMEM((2, g + dim), jnp.float32),
            pltpu.VMEM((_CHUNK_ROWS,), jnp.float32),
            pltpu.SemaphoreType.DMA((2,)),
        ],
    )(idx_r, val_r, wlin, wemb_rm)
    return out.reshape(batch, 1) + bias[None, :]


# speedup vs baseline: 4.7608x; 1.0078x over previous
import functools

import jax
import jax.numpy as jnp
from jax import lax
from jax.experimental import pallas as pl
from jax.experimental.pallas import tpu as pltpu
from jax.experimental.pallas import tpu_sc as plsc

_NC = 2
_NS = 16
_NW = _NC * _NS

_CHUNK_ROWS = 64
_GW = 128


def _fm_body(idx_hbm, val_hbm, wlin_hbm, wemb_hbm, out_hbm,
             idx_v, idx2_v, val_v, emb_v, lin_v, out_v, sems,
             *, n_chunks, n_groups, fields, dim):
    wid = lax.axis_index("s") * _NC + lax.axis_index("c")
    g = _CHUNK_ROWS * fields

    for buf in range(2):
        val_v[buf, pl.ds(g, dim)] = jnp.zeros((dim,), jnp.float32)
        lin_v[buf, pl.ds(g, dim)] = jnp.zeros((dim,), jnp.float32)

    def start_chunk(c, buf):
        pltpu.sync_copy(idx_hbm.at[wid, c], idx_v.at[buf])
        pltpu.sync_copy(val_hbm.at[wid, c], val_v.at[buf, pl.ds(0, g)])
        copies = []
        for j in range(n_groups):
            copies.append(pltpu.async_copy(
                wlin_hbm.at[idx_v.at[buf, j]],
                lin_v.at[buf, pl.ds(j * _GW, _GW)], sems.at[buf]))

        def remap_body(t, _):
            jj = t >> 3
            off = (t & 7) * dim
            x = idx_v[buf, jj, pl.ds(off, dim)]
            y = ((x & -(8 * _TR_S)) + 8 * (x & (_TR_S - 1))
                 + ((x & (8 * _TR_S - 1)) >> jnp.int32(_TR_S.bit_length()
                                                      - 1)))
            idx2_v[buf, jj, pl.ds(off, dim)] = y
            return 0

        lax.fori_loop(0, n_groups * (_GW // dim), remap_body, 0,
                      unroll=False)
        for j in range(n_groups):
            copies.append(pltpu.async_copy(
                wemb_hbm.at[idx2_v.at[buf, j]],
                emb_v.at[buf, pl.ds(j * _GW, _GW)], sems.at[buf]))
        return copies

    def compute_chunk(c, buf):
        def row_body(b, _):
            lanes = lax.iota(jnp.int32, dim)
            last_lane = lanes == (dim - 1)
            lin_mask = lanes < (fields - 16)
            n0 = b * fields
            vv1 = val_v[buf, pl.ds(n0, dim)]
            vv2 = val_v[buf, pl.ds(n0 + dim, dim)]
            lv1 = lin_v[buf, pl.ds(n0, dim)]
            lv2 = lin_v[buf, pl.ds(n0 + dim, dim)]
            acc = jnp.zeros((dim,), jnp.float32)
            accq = jnp.zeros((dim,), jnp.float32)
            for f in range(fields):
                e = emb_v[buf, n0 + f, :]
                v = vv1[f] if f < dim else vv2[f - dim]
                ev = e * v
                acc = acc + ev
                accq = accq + ev * ev
            r = (0.5 * (acc * acc - accq) + vv1 * lv1
                 + jnp.where(lin_mask, vv2 * lv2, 0.0))
            rs = plsc.cumsum(r)
            plsc.store_scatter(out_v, [jnp.full((dim,), b, jnp.int32)], rs,
                               mask=last_lane)
            return 0

        lax.fori_loop(0, _CHUNK_ROWS, row_body, 0, unroll=False)
        pltpu.sync_copy(out_v, out_hbm.at[wid, pl.ds(c * _CHUNK_ROWS,
                                                     _CHUNK_ROWS)])

    copies = start_chunk(0, 0)
    for c in range(n_chunks):
        buf = c % 2
        for cp in copies:
            cp.wait()
        if c + 1 < n_chunks:
            copies = start_chunk(c + 1, 1 - buf)
        compute_chunk(c, buf)


_TR_S = 16384


def _repack_block(wt_ref, wl_ref, out_ref, out_lin_ref):
    x = wt_ref[...]
    xcat = jnp.concatenate(
        [x[:, k * _TR_S:(k + 1) * _TR_S] for k in range(8)], axis=0)
    out_ref[...] = xcat.T
    out_lin_ref[...] = wl_ref[...].reshape(out_lin_ref.shape)


def _repack(W_embed, W_linear):
    n, dim = W_embed.shape
    wt = W_embed.T
    wl = W_linear.T
    bs = 8 * _TR_S
    grid = (n + bs - 1) // bs
    lpb = bs // (8 * dim)
    out, out_lin = pl.pallas_call(
        _repack_block,
        grid=(grid,),
        in_specs=[pl.BlockSpec((dim, bs), lambda i: (0, i)),
                  pl.BlockSpec((1, bs), lambda i: (0, i))],
        out_specs=[pl.BlockSpec((_TR_S, 8 * dim), lambda i: (i, 0)),
                   pl.BlockSpec((lpb, 8 * dim), lambda i: (i, 0))],
        out_shape=[jax.ShapeDtypeStruct((grid * _TR_S, 8 * dim),
                                        jnp.float32),
                   jax.ShapeDtypeStruct((grid * lpb, 8 * dim),
                                        jnp.float32)],
    )(wt, wl)
    return out.reshape(grid * bs, dim), out_lin.reshape(-1)


def kernel(feature_idx, feature_value, W_linear, bias, W_embed):
    batch, fields = feature_idx.shape
    dim = W_embed.shape[1]
    assert dim == 16 and 16 < fields <= 32
    assert batch % (_NW * _CHUNK_ROWS) == 0
    rows_per_w = batch // _NW
    n_chunks = rows_per_w // _CHUNK_ROWS
    g = _CHUNK_ROWS * fields
    assert g % _GW == 0
    n_groups = g // _GW

    idx_r = feature_idx.reshape(_NW, n_chunks, n_groups, _GW)
    val_r = feature_value.reshape(_NW, n_chunks, g)
    wemb_rm, wlin = _repack(W_embed, W_linear)

    mesh = plsc.VectorSubcoreMesh(core_axis_name="c", subcore_axis_name="s")
    body = functools.partial(_fm_body, n_chunks=n_chunks, n_groups=n_groups,
                             fields=fields, dim=dim)
    out = pl.kernel(
        body,
        out_type=jax.ShapeDtypeStruct((_NW, rows_per_w), jnp.float32),
        mesh=mesh,
        compiler_params=pltpu.CompilerParams(use_tc_tiling_on_sc=False,
                                             needs_layout_passes=False),
        scratch_types=[
            pltpu.VMEM((2, n_groups, _GW), jnp.int32),
            pltpu.VMEM((2, n_groups, _GW), jnp.int32),
            pltpu.VMEM((2, g + dim), jnp.float32),
            pltpu.VMEM((2, g, dim), jnp.float32),
            pltpu.VMEM((2, g + dim), jnp.float32),
            pltpu.VMEM((_CHUNK_ROWS,), jnp.float32),
            pltpu.SemaphoreType.DMA((2,)),
        ],
    )(idx_r, val_r, wlin, wemb_rm)
    return out.reshape(batch, 1) + bias[None, :]
